# baseline (speedup 1.0000x reference)
import functools

import jax
import jax.numpy as jnp
from jax import lax
from jax.experimental import pallas as pl
from jax.experimental.pallas import tpu as pltpu
from jax.experimental.pallas import tpu_sc as plsc

D = 64
S = 65
V = 38
B = 16384
NC, NS, L = 2, 16, 16
NDG = 4
NBG = 8
CB = B // NBG
VP = 40
FW = S * VP * L


def _body(in_hbm, token_hbm, pos_hbm, out_hbm,
          token_v, pos_v, fused, idxrow, slab0, slab1, osem0, osem1):
    cid = lax.axis_index("c")
    sid = lax.axis_index("s")
    wid = sid * NC + cid
    d0 = (wid % NDG) * L
    b0 = (wid // NDG) * CB

    pltpu.sync_copy(token_hbm, token_v)
    pltpu.sync_copy(pos_hbm, pos_v)

    def build_s(s, carry):
        pv = pos_v[pl.ds(s * D + d0, L)]
        for v in range(V):
            fused[pl.ds(s * (VP * L) + v * L, L)] = (
                token_v[pl.ds(v * D + d0, L)] + pv)
        return carry

    lax.fori_loop(0, S, build_s, 0)

    def emit_slab(s, slab, osem, first):
        pltpu.sync_copy(in_hbm.at[pl.ds(s * B + b0, CB)], idxrow)
        sbase = s * (VP * L)
        if not first:
            pltpu.make_async_copy(
                slab, out_hbm.at[pl.ds(0, 1), pl.ds(0, L), pl.ds(0, CB)],
                osem).wait()

        def cbody(c, carry):
            raw = idxrow[pl.ds(c * L, L)]
            rawb = raw * L + sbase
            for dd in range(L):
                val = plsc.load_gather(fused, [rawb + dd])
                slab[0, dd, pl.ds(c * L, L)] = val
            return carry

        lax.fori_loop(0, CB // L, cbody, 0)
        pltpu.async_copy(
            slab, out_hbm.at[pl.ds(s, 1), pl.ds(d0, L), pl.ds(b0, CB)],
            osem)

    emit_slab(S - 1, slab0, osem0, True)
    emit_slab(S - 2, slab1, osem1, True)

    def pair_body(k, carry):
        emit_slab(S - 3 - 2 * k, slab0, osem0, False)
        emit_slab(S - 4 - 2 * k, slab1, osem1, False)
        return carry

    lax.fori_loop(0, (S - 3) // 2, pair_body, 0)
    emit_slab(0, slab0, osem0, False)

    for slab, osem in ((slab0, osem0), (slab1, osem1)):
        pltpu.make_async_copy(
            slab, out_hbm.at[pl.ds(0, 1), pl.ds(0, L), pl.ds(0, CB)],
            osem).wait()


@jax.jit
def kernel(inputs, token_table, pos_table):
    mesh = plsc.VectorSubcoreMesh(
        core_axis_name="c", subcore_axis_name="s",
        num_cores=NC, num_subcores=NS)
    run = functools.partial(
        pl.kernel,
        out_type=jax.ShapeDtypeStruct((S, D, B), jnp.float32),
        mesh=mesh,
        scratch_types=[
            pltpu.VMEM((V * D,), jnp.float32),
            pltpu.VMEM((S * D,), jnp.float32),
            pltpu.VMEM((FW,), jnp.float32),
            pltpu.VMEM((CB,), jnp.int32),
            pltpu.VMEM((1, L, CB), jnp.float32),
            pltpu.VMEM((1, L, CB), jnp.float32),
            pltpu.SemaphoreType.DMA,
            pltpu.SemaphoreType.DMA,
        ],
        compiler_params=pltpu.CompilerParams(
            use_tc_tiling_on_sc=False, needs_layout_passes=False),
    )(_body)
    out_t = run(inputs.T.reshape(S * B), token_table.reshape(V * D),
                pos_table.reshape(S * D))
    return out_t.transpose(2, 0, 1)


# speedup vs baseline: 2.0002x; 2.0002x over previous
import functools

import jax
import jax.numpy as jnp
from jax import lax
from jax.experimental import pallas as pl
from jax.experimental.pallas import tpu as pltpu
from jax.experimental.pallas import tpu_sc as plsc

D = 64
S = 65
V = 38
B = 16384
NC, NS, L = 2, 16, 16
NDG = 4
NBG = 8
CB = B // NBG
VP = 40
FW = S * VP * L


def _body(in_hbm, token_hbm, pos_hbm, out_hbm,
          token_v, pos_v, fused, idxrow, slab0, slab1, osem0, osem1):
    cid = lax.axis_index("c")
    sid = lax.axis_index("s")
    wid = sid * NC + cid
    d0 = (wid % NDG) * L
    b0 = (wid // NDG) * CB

    pltpu.sync_copy(token_hbm, token_v)
    pltpu.sync_copy(pos_hbm, pos_v)

    def build_s(s, carry):
        pv = pos_v[pl.ds(s * D + d0, L)]
        for v in range(V):
            fused[pl.ds(s * (VP * L) + v * L, L)] = (
                token_v[pl.ds(v * D + d0, L)] + pv)
        return carry

    lax.fori_loop(0, S, build_s, 0)

    def emit_slab(s, slab, osem, first):
        pltpu.sync_copy(in_hbm.at[pl.ds(s * B + b0, CB)], idxrow)
        sbase = s * (VP * L)
        if not first:
            pltpu.make_async_copy(
                slab, out_hbm.at[pl.ds(0, 1), pl.ds(0, L), pl.ds(0, CB)],
                osem).wait()

        @plsc.parallel_loop(0, CB // L, unroll=2)
        def cbody(c):
            raw = idxrow[pl.ds(c * L, L)]
            rawb = raw * L + sbase
            for dd in range(L):
                val = plsc.load_gather(fused, [rawb + dd])
                slab[0, dd, pl.ds(c * L, L)] = val
        pltpu.async_copy(
            slab, out_hbm.at[pl.ds(s, 1), pl.ds(d0, L), pl.ds(b0, CB)],
            osem)

    emit_slab(S - 1, slab0, osem0, True)
    emit_slab(S - 2, slab1, osem1, True)

    def pair_body(k, carry):
        emit_slab(S - 3 - 2 * k, slab0, osem0, False)
        emit_slab(S - 4 - 2 * k, slab1, osem1, False)
        return carry

    lax.fori_loop(0, (S - 3) // 2, pair_body, 0)
    emit_slab(0, slab0, osem0, False)

    for slab, osem in ((slab0, osem0), (slab1, osem1)):
        pltpu.make_async_copy(
            slab, out_hbm.at[pl.ds(0, 1), pl.ds(0, L), pl.ds(0, CB)],
            osem).wait()


@jax.jit
def kernel(inputs, token_table, pos_table):
    mesh = plsc.VectorSubcoreMesh(
        core_axis_name="c", subcore_axis_name="s",
        num_cores=NC, num_subcores=NS)
    run = functools.partial(
        pl.kernel,
        out_type=jax.ShapeDtypeStruct((S, D, B), jnp.float32),
        mesh=mesh,
        scratch_types=[
            pltpu.VMEM((V * D,), jnp.float32),
            pltpu.VMEM((S * D,), jnp.float32),
            pltpu.VMEM((FW,), jnp.float32),
            pltpu.VMEM((CB,), jnp.int32),
            pltpu.VMEM((1, L, CB), jnp.float32),
            pltpu.VMEM((1, L, CB), jnp.float32),
            pltpu.SemaphoreType.DMA,
            pltpu.SemaphoreType.DMA,
        ],
        compiler_params=pltpu.CompilerParams(
            use_tc_tiling_on_sc=False, needs_layout_passes=False),
    )(_body)
    out_t = run(inputs.T.reshape(S * B), token_table.reshape(V * D),
                pos_table.reshape(S * D))
    return out_t.transpose(2, 0, 1)
